# baseline (speedup 1.0000x reference)
import math

import jax
import jax.numpy as jnp
from jax.experimental import pallas as pl
from jax.experimental.pallas import tpu as pltpu

B, L, D, H = 2, 2048, 1024, 16
HD = D // H
NPAIR = H // 2
K_TOP = math.ceil(math.sqrt(L))
KSEL = 48
SCALE = 1.0 / math.sqrt(HD)

F32 = jnp.float32
BF16 = jnp.bfloat16


def _mega_kernel(x_ref, wq_ref, bq_ref, wkv_ref, bkv_ref, ow_ref, ob_ref,
                 selw_ref, y_ref):
    x = x_ref[0]
    xb = x.astype(BF16)

    col_logit = jnp.sum(x * selw_ref[...], axis=1, keepdims=True)
    vals0 = jnp.transpose(col_logit)
    pos = jax.lax.broadcasted_iota(jnp.int32, (1, L), 1)
    lanek = jax.lax.broadcasted_iota(jnp.int32, (1, KSEL), 1)

    def body(i, carry):
        vals, idxv = carry
        m = jnp.max(vals, axis=1, keepdims=True)
        cand = jnp.where(vals == m, pos, L)
        jv = jnp.min(cand, axis=1, keepdims=True)
        idxv = jnp.where(lanek == i, jv, idxv)
        vals = jnp.where(pos == jv, -1e30, vals)
        return vals, idxv

    idx0 = jnp.zeros((1, KSEL), jnp.int32)
    _, idxv = jax.lax.fori_loop(0, K_TOP, body, (vals0, idx0))

    eye_r = jax.lax.broadcasted_iota(jnp.int32, (KSEL, KSEL), 0)
    eye_c = jax.lax.broadcasted_iota(jnp.int32, (KSEL, KSEL), 1)
    eye = (eye_r == eye_c).astype(F32)
    idx_f = idxv.astype(F32)
    t_col = jax.lax.dot_general(eye, idx_f, (((1,), (1,)), ((), ())),
                                preferred_element_type=F32)
    col_f = jax.lax.broadcasted_iota(jnp.int32, (KSEL, L), 1).astype(F32)
    p_gather = (col_f == t_col).astype(F32)
    x_sel = jax.lax.dot_general(p_gather, x, (((1,), (0,)), ((), ())),
                                preferred_element_type=F32)

    causal = jnp.where(col_f <= t_col, 0.0, -10000.0)
    lane128 = jax.lax.broadcasted_iota(jnp.int32, (1, 128), 1)
    m0 = (lane128 < HD).astype(F32)
    m1 = 1.0 - m0
    masks = (m0, m1)
    maskbs = (m0.astype(BF16), m1.astype(BF16))
    xsb = x_sel.astype(BF16)
    parts = []
    for g in range(NPAIR):
        kv_chunk = jax.lax.dot_general(
            xb, wkv_ref[g], (((1,), (1,)), ((), ())),
            preferred_element_type=F32) + bkv_ref[g]
        kvb = kv_chunk.astype(BF16)
        kp = kvb[:, :128]
        vp = kvb[:, 128:]
        q = jax.lax.dot_general(xsb, wq_ref[g], (((1,), (1,)), ((), ())),
                                preferred_element_type=F32) + bq_ref[g]
        ctx_pair = jnp.zeros((KSEL, 128), F32)
        for h in range(2):
            qb = (q * masks[h]).astype(BF16)
            s = jax.lax.dot_general(qb, kp, (((1,), (1,)), ((), ())),
                                    preferred_element_type=F32)
            s = s * SCALE + causal
            s = s - jnp.max(s, axis=1, keepdims=True)
            p = jnp.exp(s)
            p = p / jnp.sum(p, axis=1, keepdims=True)
            ctx_pair = ctx_pair + jax.lax.dot_general(
                p.astype(BF16), vp * maskbs[h], (((1,), (0,)), ((), ())),
                preferred_element_type=F32)
        parts.append(ctx_pair)
    ctx = jnp.concatenate(parts, axis=1)

    attn = jax.lax.dot_general(ctx.astype(BF16), ow_ref[...],
                               (((1,), (1,)), ((), ())),
                               preferred_element_type=F32) + ob_ref[...]
    logit = jnp.sum(x_sel * selw_ref[...], axis=1, keepdims=True)
    src = attn * jax.nn.sigmoid(logit)

    rows = jax.lax.broadcasted_iota(jnp.int32, (L, KSEL), 0)
    cols = jax.lax.broadcasted_iota(jnp.int32, (L, KSEL), 1)
    p_sc = jnp.logical_and(rows == idxv, cols < K_TOP).astype(BF16)
    y_ref[0] = x + jax.lax.dot_general(
        p_sc, src.astype(BF16), (((1,), (0,)), ((), ())),
        preferred_element_type=F32)


@jax.jit
def kernel(x, Wqkv_w, Wqkv_b, sel_w, out_w, out_b):
    wq = Wqkv_w[:D]
    wk3 = Wqkv_w[D:2 * D].reshape(NPAIR, 128, D)
    wv3 = Wqkv_w[2 * D:].reshape(NPAIR, 128, D)
    bq3 = Wqkv_b[:D].reshape(NPAIR, 1, 128)
    bk3 = Wqkv_b[D:2 * D].reshape(NPAIR, 1, 128)
    bv3 = Wqkv_b[2 * D:].reshape(NPAIR, 1, 128)
    bkv3 = jnp.concatenate([bk3, bv3], axis=2)
    selw = sel_w.reshape(1, D)
    obr = out_b.reshape(1, D)
    wq3b = wq.reshape(NPAIR, 128, D).astype(BF16)
    wkv3b = jnp.concatenate([wk3, wv3], axis=1).astype(BF16)
    out_wb = out_w.astype(BF16)

    return pl.pallas_call(
        _mega_kernel,
        grid=(B,),
        in_specs=[
            pl.BlockSpec((1, L, D), lambda b: (b, 0, 0)),
            pl.BlockSpec((NPAIR, 128, D), lambda b: (0, 0, 0)),
            pl.BlockSpec((NPAIR, 1, 128), lambda b: (0, 0, 0)),
            pl.BlockSpec((NPAIR, 256, D), lambda b: (0, 0, 0)),
            pl.BlockSpec((NPAIR, 1, 256), lambda b: (0, 0, 0)),
            pl.BlockSpec((D, D), lambda b: (0, 0)),
            pl.BlockSpec((1, D), lambda b: (0, 0)),
            pl.BlockSpec((1, D), lambda b: (0, 0)),
        ],
        out_specs=pl.BlockSpec((1, L, D), lambda b: (b, 0, 0)),
        out_shape=jax.ShapeDtypeStruct((B, L, D), F32),
    )(x, wq3b, bq3, wkv3b, bkv3, out_wb, obr, selw)


# speedup vs baseline: 1.1355x; 1.1355x over previous
import math

import jax
import jax.numpy as jnp
from jax.experimental import pallas as pl
from jax.experimental.pallas import tpu as pltpu

B, L, D, H = 2, 2048, 1024, 16
HD = D // H
NPAIR = H // 2
K_TOP = math.ceil(math.sqrt(L))
KSEL = 48
LBLK = 512
NL = L // LBLK
MBLK = 512
SCALE = 1.0 / math.sqrt(HD)
LOG2E = math.log2(math.e)

F32 = jnp.float32
BF16 = jnp.bfloat16


def _select_kernel(x_ref, selw_ref, idx_ref, t_ref, xsel_ref):
    rows = []
    for b in range(B):
        col = jnp.sum(x_ref[b] * selw_ref[...], axis=1, keepdims=True)
        rows.append(jnp.transpose(col).reshape(1, 1, L))
    vals0 = jnp.concatenate(rows, axis=0)
    pos = jax.lax.broadcasted_iota(jnp.int32, (B, 1, L), 2)
    lanek = jax.lax.broadcasted_iota(jnp.int32, (B, 1, KSEL), 2)

    def body(i, carry):
        vals, idxv = carry
        m = jnp.max(vals, axis=2, keepdims=True)
        cand = jnp.where(vals == m, pos, L)
        jv = jnp.min(cand, axis=2, keepdims=True)
        idxv = jnp.where(lanek == i, jv, idxv)
        vals = jnp.where(pos == jv, -1e30, vals)
        return vals, idxv

    idx0 = jnp.zeros((B, 1, KSEL), jnp.int32)
    _, idxv = jax.lax.fori_loop(0, K_TOP, body, (vals0, idx0))
    idx_ref[...] = idxv

    eye_r = jax.lax.broadcasted_iota(jnp.int32, (KSEL, KSEL), 0)
    eye_c = jax.lax.broadcasted_iota(jnp.int32, (KSEL, KSEL), 1)
    eye = (eye_r == eye_c).astype(F32)
    idx_f = idxv.astype(F32)
    col_f = jax.lax.broadcasted_iota(jnp.int32, (KSEL, L), 1).astype(F32)
    for b in range(B):
        t_col = jax.lax.dot_general(eye, idx_f[b], (((1,), (1,)), ((), ())),
                                    preferred_element_type=F32)
        t_ref[b] = t_col
        p = (col_f == t_col).astype(F32)
        xsel_ref[b] = jax.lax.dot_general(
            p, x_ref[b], (((1,), (0,)), ((), ())), preferred_element_type=F32)


def _select(x, selw):
    return pl.pallas_call(
        _select_kernel,
        grid=(1,),
        in_specs=[
            pl.BlockSpec((B, L, D), lambda i: (0, 0, 0)),
            pl.BlockSpec((1, D), lambda i: (0, 0)),
        ],
        out_specs=[
            pl.BlockSpec((B, 1, KSEL), lambda i: (0, 0, 0)),
            pl.BlockSpec((B, KSEL, 1), lambda i: (0, 0, 0)),
            pl.BlockSpec((B, KSEL, D), lambda i: (0, 0, 0)),
        ],
        out_shape=[
            jax.ShapeDtypeStruct((B, 1, KSEL), jnp.int32),
            jax.ShapeDtypeStruct((B, KSEL, 1), F32),
            jax.ShapeDtypeStruct((B, KSEL, D), F32),
        ],
    )(x, selw)


def _fused_attn_kernel(x_ref, wq_ref, bq_ref, wkv_ref, bkv_ref, ow_ref,
                       ob_ref, selw_ref, xs_ref, t_ref, src_ref,
                       qs_ref, kv_ref):
    lblk = pl.program_id(1)

    @pl.when(lblk == 0)
    def _init():
        xsb = xs_ref[0].astype(BF16)
        for g in range(NPAIR):
            qs_ref[g] = jax.lax.dot_general(
                xsb, wq_ref[g], (((1,), (1,)), ((), ())),
                preferred_element_type=F32) + bq_ref[g]

    xb = x_ref[0].astype(BF16)
    kv = jax.lax.dot_general(xb, wkv_ref[...], (((1,), (1,)), ((), ())),
                             preferred_element_type=F32) + bkv_ref[...]
    kvb = kv.astype(BF16)
    for g in range(2 * NPAIR):
        kv_ref[g, pl.ds(lblk * LBLK, LBLK), :] = kvb[:, 128 * g:128 * (g + 1)]

    @pl.when(lblk == NL - 1)
    def _finalize():
        t = t_ref[0]
        colp = jax.lax.broadcasted_iota(jnp.int32, (KSEL, L), 1).astype(F32)
        causal2 = jnp.where(colp <= t, 0.0, -10000.0 * LOG2E)
        lane128 = jax.lax.broadcasted_iota(jnp.int32, (1, 128), 1)
        m0 = (lane128 < HD).astype(F32)
        m1 = 1.0 - m0
        qscales = (m0 * (SCALE * LOG2E), m1 * (SCALE * LOG2E))
        maskbs = (m0.astype(BF16), m1.astype(BF16))
        parts = []
        for g in range(NPAIR):
            kp = kv_ref[g]
            vp = kv_ref[NPAIR + g]
            q = qs_ref[g]
            ctx_pair = jnp.zeros((KSEL, 128), F32)
            ls = []
            for h in range(2):
                qb = (q * qscales[h]).astype(BF16)
                s2 = jax.lax.dot_general(qb, kp, (((1,), (1,)), ((), ())),
                                         preferred_element_type=F32)
                s2 = s2 + causal2
                p = jnp.exp2(s2 - jnp.max(s2, axis=1, keepdims=True))
                ls.append(jnp.sum(p, axis=1, keepdims=True))
                ctx_pair = ctx_pair + jax.lax.dot_general(
                    p.astype(BF16), vp * maskbs[h], (((1,), (0,)), ((), ())),
                    preferred_element_type=F32)
            denom = ls[0] * m0 + ls[1] * m1
            parts.append(ctx_pair / denom)
        ctx = jnp.concatenate(parts, axis=1)
        attn = jax.lax.dot_general(ctx.astype(BF16), ow_ref[...],
                                   (((1,), (1,)), ((), ())),
                                   preferred_element_type=F32) + ob_ref[...]
        logit = jnp.sum(xs_ref[0] * selw_ref[...], axis=1, keepdims=True)
        src_ref[0] = attn * jax.nn.sigmoid(logit)


def _fused_attn(x, wq3b, bq3, wkvb, bkv, out_wb, obr, selw, x_sel, t_col):
    return pl.pallas_call(
        _fused_attn_kernel,
        grid=(B, NL),
        in_specs=[
            pl.BlockSpec((1, LBLK, D), lambda b, i: (b, i, 0)),
            pl.BlockSpec((NPAIR, 128, D), lambda b, i: (0, 0, 0)),
            pl.BlockSpec((NPAIR, 1, 128), lambda b, i: (0, 0, 0)),
            pl.BlockSpec((2 * D, D), lambda b, i: (0, 0)),
            pl.BlockSpec((1, 2 * D), lambda b, i: (0, 0)),
            pl.BlockSpec((D, D), lambda b, i: (0, 0)),
            pl.BlockSpec((1, D), lambda b, i: (0, 0)),
            pl.BlockSpec((1, D), lambda b, i: (0, 0)),
            pl.BlockSpec((1, KSEL, D), lambda b, i: (b, 0, 0)),
            pl.BlockSpec((1, KSEL, 1), lambda b, i: (b, 0, 0)),
        ],
        out_specs=pl.BlockSpec((1, KSEL, D), lambda b, i: (b, 0, 0)),
        out_shape=jax.ShapeDtypeStruct((B, KSEL, D), F32),
        scratch_shapes=[
            pltpu.VMEM((NPAIR, KSEL, 128), F32),
            pltpu.VMEM((2 * NPAIR, L, 128), BF16),
        ],
    )(x, wq3b, bq3, wkvb, bkv, out_wb, obr, selw, x_sel, t_col)


def _merge_kernel(x_ref, src_ref, idx_ref, y_ref):
    base = pl.program_id(1) * MBLK
    rows = jax.lax.broadcasted_iota(jnp.int32, (MBLK, KSEL), 0) + base
    cols = jax.lax.broadcasted_iota(jnp.int32, (MBLK, KSEL), 1)
    idxr = idx_ref[0]
    p = jnp.logical_and(rows == idxr, cols < K_TOP).astype(BF16)
    y_ref[0] = x_ref[0] + jax.lax.dot_general(
        p, src_ref[0].astype(BF16), (((1,), (0,)), ((), ())),
        preferred_element_type=F32)


def _merge(x, src, idx):
    nm = L // MBLK
    return pl.pallas_call(
        _merge_kernel,
        grid=(B, nm),
        in_specs=[
            pl.BlockSpec((1, MBLK, D), lambda b, i: (b, i, 0)),
            pl.BlockSpec((1, KSEL, D), lambda b, i: (b, 0, 0)),
            pl.BlockSpec((1, 1, KSEL), lambda b, i: (b, 0, 0)),
        ],
        out_specs=pl.BlockSpec((1, MBLK, D), lambda b, i: (b, i, 0)),
        out_shape=jax.ShapeDtypeStruct((B, L, D), F32),
    )(x, src, idx)


@jax.jit
def kernel(x, Wqkv_w, Wqkv_b, sel_w, out_w, out_b):
    wq = Wqkv_w[:D]
    wkv = Wqkv_w[D:]
    bq3 = Wqkv_b[:D].reshape(NPAIR, 1, 128)
    bkv = Wqkv_b[D:].reshape(1, 2 * D)
    selw = sel_w.reshape(1, D)
    obr = out_b.reshape(1, D)
    wq3b = wq.reshape(NPAIR, 128, D).astype(BF16)
    wkvb = wkv.astype(BF16)
    out_wb = out_w.astype(BF16)

    idx, t_col, x_sel = _select(x, selw)
    src = _fused_attn(x, wq3b, bq3, wkvb, bkv, out_wb, obr, selw, x_sel,
                      t_col)
    return _merge(x, src, idx)
